# baseline (speedup 1.0000x reference)
import functools

import jax
import jax.numpy as jnp
from jax import lax
from jax.experimental import pallas as pl
from jax.experimental.pallas import tpu as pltpu
from jax.experimental.pallas import tpu_sc as plsc

RANK = 64
CHUNKS = 2
NUM_NODES = 262144
B = 256
N = 128
ROWS = B * N
D = RANK * CHUNKS
HID = 10 * RANK

NC = 2
NS = 16
NW = NC * NS
ROWS_PER_W = ROWS // NW
GCHUNK = 512
N_GCHUNKS = ROWS_PER_W // GCHUNK


def _sc_gather_body(idx_hbm, table_hbm, mask_hbm, emb_out, mask_out,
                    idx_v, rows_v, mval_v, sem, sem2):
    wid = lax.axis_index("s") * NC + lax.axis_index("c")
    base = wid * ROWS_PER_W
    for c in range(N_GCHUNKS):
        off = base + c * GCHUNK
        pltpu.sync_copy(idx_hbm.at[pl.ds(off, GCHUNK)], idx_v)
        row_cp = pltpu.async_copy(table_hbm.at[idx_v], rows_v, sem)
        msk_cp = pltpu.async_copy(mask_hbm.at[idx_v], mval_v, sem2)
        row_cp.wait()
        msk_cp.wait()
        pltpu.sync_copy(rows_v, emb_out.at[pl.ds(off, GCHUNK)])
        pltpu.sync_copy(mval_v, mask_out.at[pl.ds(off, GCHUNK)])


@functools.cache
def _sc_gather():
    return pl.kernel(
        _sc_gather_body,
        out_type=(
            jax.ShapeDtypeStruct((ROWS, D), jnp.float32),
            jax.ShapeDtypeStruct((ROWS,), jnp.int32),
        ),
        mesh=plsc.VectorSubcoreMesh(
            core_axis_name="c", subcore_axis_name="s",
            num_cores=NC, num_subcores=NS,
        ),
        scratch_types=[
            pltpu.VMEM((GCHUNK,), jnp.int32),
            pltpu.VMEM((GCHUNK, D), jnp.float32),
            pltpu.VMEM((GCHUNK,), jnp.int32),
            pltpu.SemaphoreType.DMA,
            pltpu.SemaphoreType.DMA,
        ],
        compiler_params=pltpu.CompilerParams(use_tc_tiling_on_sc=True),
    )


def _mlp_body(e_ref, qc_ref, m_ref, w1a_ref, w1b_ref, w2a_ref, w2b_ref,
              b1_ref, b2_ref, o_ref, x_ref, w1s, w2s):
    bs = qc_ref.shape[0]

    @pl.when(pl.program_id(0) == 0)
    def _pack():
        zb = jnp.zeros((RANK, HID), jnp.bfloat16)
        w1s[0:RANK, 0:HID] = w1a_ref[0:RANK, :].astype(jnp.bfloat16)
        w1s[RANK:D, 0:HID] = zb
        w1s[D:2 * D, 0:HID] = w1a_ref[RANK:3 * RANK, :].astype(jnp.bfloat16)
        w1s[0:RANK, HID:2 * HID] = zb
        w1s[RANK:D, HID:2 * HID] = w1b_ref[0:RANK, :].astype(jnp.bfloat16)
        w1s[D:2 * D, HID:2 * HID] = w1b_ref[RANK:3 * RANK, :].astype(jnp.bfloat16)
        zc = jnp.zeros((HID, RANK), jnp.bfloat16)
        w2s[0:HID, 0:RANK] = w2a_ref[...].astype(jnp.bfloat16)
        w2s[0:HID, RANK:D] = zc
        w2s[HID:2 * HID, 0:RANK] = zc
        w2s[HID:2 * HID, RANK:D] = w2b_ref[...].astype(jnp.bfloat16)

    x_ref[:, 0:D] = e_ref[...].astype(jnp.bfloat16)
    for j in range(bs):
        x_ref[pl.ds(j * N, N), D:2 * D] = qc_ref[j, :, :].astype(jnp.bfloat16)
    h = jnp.dot(x_ref[...], w1s[...], preferred_element_type=jnp.float32)
    h = jnp.maximum(h + b1_ref[...], 0.0).astype(jnp.bfloat16)
    y = jnp.dot(h, w2s[...], preferred_element_type=jnp.float32)
    y = y + b2_ref[...]
    nonleaf = m_ref[...] == 0
    out = jnp.where(nonleaf, y, e_ref[...])
    for j in range(bs):
        o_ref[j, :, :] = out[j * N:(j + 1) * N, :]


def _run_mlp(embeds, qcat, mask2d, w1a, w1b, w2a, w2b, b1p, b2p, bs=8):
    grid = B // bs
    r = bs * N
    xin = 2 * D
    res = lambda i: (0, 0)
    return pl.pallas_call(
        _mlp_body,
        grid=(grid,),
        in_specs=[
            pl.BlockSpec((r, D), lambda i: (i, 0)),
            pl.BlockSpec((bs, N, D), lambda i: (i, 0, 0)),
            pl.BlockSpec((r, 1), lambda i: (i, 0)),
            pl.BlockSpec((3 * RANK, HID), res),
            pl.BlockSpec((3 * RANK, HID), res),
            pl.BlockSpec((HID, RANK), res),
            pl.BlockSpec((HID, RANK), res),
            pl.BlockSpec((1, 2 * HID), res),
            pl.BlockSpec((1, D), res),
        ],
        out_specs=pl.BlockSpec((bs, N, D), lambda i: (i, 0, 0)),
        out_shape=jax.ShapeDtypeStruct((B, N, D), jnp.float32),
        scratch_shapes=[
            pltpu.VMEM((r, xin), jnp.bfloat16),
            pltpu.VMEM((xin, 2 * HID), jnp.bfloat16),
            pltpu.VMEM((2 * HID, D), jnp.bfloat16),
        ],
    )(embeds, qcat, mask2d, w1a, w1b, w2a, w2b, b1p, b2p)


def kernel(nodeIdx, leaf_mask, q0, q1, table,
           qt0_w1, qt0_b1, qt0_w2, qt0_b2, qt1_w1, qt1_b1, qt1_w2, qt1_b2):
    flat_idx = nodeIdx.reshape(ROWS)
    embeds, mask = _sc_gather()(flat_idx, table, leaf_mask)
    b1p = jnp.concatenate([qt0_b1, qt1_b1]).reshape(1, 2 * HID)
    b2p = jnp.concatenate([qt0_b2, qt1_b2]).reshape(1, D)
    qcat = jnp.concatenate([q0, q1], axis=-1)
    out = _run_mlp(embeds, qcat, mask.reshape(ROWS, 1),
                   qt0_w1, qt1_w1, qt0_w2, qt1_w2, b1p, b2p)
    out0 = lax.slice(out, (0, 0, 0), (B, N, RANK))
    out1 = lax.slice(out, (0, 0, RANK), (B, N, D))
    return out0, out1


# speedup vs baseline: 1.0632x; 1.0632x over previous
import functools

import jax
import jax.numpy as jnp
from jax import lax
from jax.experimental import pallas as pl
from jax.experimental.pallas import tpu as pltpu
from jax.experimental.pallas import tpu_sc as plsc

RANK = 64
CHUNKS = 2
NUM_NODES = 262144
B = 256
N = 128
ROWS = B * N
D = RANK * CHUNKS
HID = 10 * RANK

NC = 2
NS = 16
NW = NC * NS
ROWS_PER_W = ROWS // NW
GCHUNK = 512
N_GCHUNKS = ROWS_PER_W // GCHUNK


def _sc_gather_body(idx_hbm, table_hbm, mask_hbm, emb_out, mask_out,
                    idx_v, rows_v, mval_v, sem, sem2):
    wid = lax.axis_index("s") * NC + lax.axis_index("c")
    base = wid * ROWS_PER_W
    for c in range(N_GCHUNKS):
        off = base + c * GCHUNK
        pltpu.sync_copy(idx_hbm.at[pl.ds(off, GCHUNK)], idx_v)
        row_cp = pltpu.async_copy(table_hbm.at[idx_v], rows_v, sem)
        msk_cp = pltpu.async_copy(mask_hbm.at[idx_v], mval_v, sem2)
        row_cp.wait()
        msk_cp.wait()
        pltpu.sync_copy(rows_v, emb_out.at[pl.ds(off, GCHUNK)])
        pltpu.sync_copy(mval_v, mask_out.at[pl.ds(off, GCHUNK)])


@functools.cache
def _sc_gather():
    return pl.kernel(
        _sc_gather_body,
        out_type=(
            jax.ShapeDtypeStruct((ROWS, D), jnp.float32),
            jax.ShapeDtypeStruct((ROWS,), jnp.int32),
        ),
        mesh=plsc.VectorSubcoreMesh(
            core_axis_name="c", subcore_axis_name="s",
            num_cores=NC, num_subcores=NS,
        ),
        scratch_types=[
            pltpu.VMEM((GCHUNK,), jnp.int32),
            pltpu.VMEM((GCHUNK, D), jnp.float32),
            pltpu.VMEM((GCHUNK,), jnp.int32),
            pltpu.SemaphoreType.DMA,
            pltpu.SemaphoreType.DMA,
        ],
        compiler_params=pltpu.CompilerParams(use_tc_tiling_on_sc=True),
    )


def _mlp_body(e_ref, q0_ref, q1_ref, m_ref, w1a_ref, w1b_ref, w2a_ref, w2b_ref,
              b1_ref, b2_ref, o_ref, x_ref, w1s, w2s):
    bs = q0_ref.shape[0]

    @pl.when(pl.program_id(0) == 0)
    def _pack():
        zb = jnp.zeros((RANK, HID), jnp.bfloat16)
        w1s[0:RANK, 0:HID] = w1a_ref[0:RANK, :].astype(jnp.bfloat16)
        w1s[RANK:D, 0:HID] = zb
        w1s[D:2 * D, 0:HID] = w1a_ref[RANK:3 * RANK, :].astype(jnp.bfloat16)
        w1s[0:RANK, HID:2 * HID] = zb
        w1s[RANK:D, HID:2 * HID] = w1b_ref[0:RANK, :].astype(jnp.bfloat16)
        w1s[D:2 * D, HID:2 * HID] = w1b_ref[RANK:3 * RANK, :].astype(jnp.bfloat16)
        zc = jnp.zeros((HID, RANK), jnp.bfloat16)
        w2s[0:HID, 0:RANK] = w2a_ref[...].astype(jnp.bfloat16)
        w2s[0:HID, RANK:D] = zc
        w2s[HID:2 * HID, 0:RANK] = zc
        w2s[HID:2 * HID, RANK:D] = w2b_ref[...].astype(jnp.bfloat16)

    x_ref[:, 0:D] = e_ref[...].astype(jnp.bfloat16)
    for j in range(bs):
        x_ref[pl.ds(j * N, N), D:D + RANK] = q0_ref[j, :, :].astype(jnp.bfloat16)
        x_ref[pl.ds(j * N, N), D + RANK:2 * D] = (
            q1_ref[j, :, :].astype(jnp.bfloat16))
    h = jnp.dot(x_ref[...], w1s[...], preferred_element_type=jnp.float32)
    h = jnp.maximum(h + b1_ref[...], 0.0).astype(jnp.bfloat16)
    y = jnp.dot(h, w2s[...], preferred_element_type=jnp.float32)
    y = y + b2_ref[...]
    nonleaf = m_ref[...] == 0
    out = jnp.where(nonleaf, y, e_ref[...])
    for j in range(bs):
        o_ref[j, :, :] = out[j * N:(j + 1) * N, :]


def _run_mlp(embeds, q0, q1, mask2d, w1a, w1b, w2a, w2b, b1p, b2p, bs=8):
    grid = B // bs
    r = bs * N
    xin = 2 * D
    res = lambda i: (0, 0)
    return pl.pallas_call(
        _mlp_body,
        grid=(grid,),
        in_specs=[
            pl.BlockSpec((r, D), lambda i: (i, 0)),
            pl.BlockSpec((bs, N, RANK), lambda i: (i, 0, 0)),
            pl.BlockSpec((bs, N, RANK), lambda i: (i, 0, 0)),
            pl.BlockSpec((r, 1), lambda i: (i, 0)),
            pl.BlockSpec((3 * RANK, HID), res),
            pl.BlockSpec((3 * RANK, HID), res),
            pl.BlockSpec((HID, RANK), res),
            pl.BlockSpec((HID, RANK), res),
            pl.BlockSpec((1, 2 * HID), res),
            pl.BlockSpec((1, D), res),
        ],
        out_specs=pl.BlockSpec((bs, N, D), lambda i: (i, 0, 0)),
        out_shape=jax.ShapeDtypeStruct((B, N, D), jnp.float32),
        scratch_shapes=[
            pltpu.VMEM((r, xin), jnp.bfloat16),
            pltpu.VMEM((xin, 2 * HID), jnp.bfloat16),
            pltpu.VMEM((2 * HID, D), jnp.bfloat16),
        ],
    )(embeds, q0, q1, mask2d, w1a, w1b, w2a, w2b, b1p, b2p)


def kernel(nodeIdx, leaf_mask, q0, q1, table,
           qt0_w1, qt0_b1, qt0_w2, qt0_b2, qt1_w1, qt1_b1, qt1_w2, qt1_b2):
    flat_idx = nodeIdx.reshape(ROWS)
    embeds, mask = _sc_gather()(flat_idx, table, leaf_mask)
    b1p = jnp.concatenate([qt0_b1, qt1_b1]).reshape(1, 2 * HID)
    b2p = jnp.concatenate([qt0_b2, qt1_b2]).reshape(1, D)
    out = _run_mlp(embeds, q0, q1, mask.reshape(ROWS, 1),
                   qt0_w1, qt1_w1, qt0_w2, qt1_w2, b1p, b2p)
    out0 = lax.slice(out, (0, 0, 0), (B, N, RANK))
    out1 = lax.slice(out, (0, 0, RANK), (B, N, D))
    return out0, out1


# speedup vs baseline: 1.2018x; 1.1304x over previous
import functools

import jax
import jax.numpy as jnp
from jax import lax
from jax.experimental import pallas as pl
from jax.experimental.pallas import tpu as pltpu
from jax.experimental.pallas import tpu_sc as plsc

RANK = 64
CHUNKS = 2
NUM_NODES = 262144
B = 256
N = 128
ROWS = B * N
D = RANK * CHUNKS
HID = 10 * RANK

NC = 2
NS = 16
NW = NC * NS
ROWS_PER_W = ROWS // NW
GCHUNK = 512
N_GCHUNKS = ROWS_PER_W // GCHUNK


def _sc_gather_body(idx_hbm, table_hbm, mask_hbm, emb_out, mask_out,
                    idx_v, rows_v, mval_v, sem, sem2):
    wid = lax.axis_index("s") * NC + lax.axis_index("c")
    base = wid * ROWS_PER_W
    for c in range(N_GCHUNKS):
        off = base + c * GCHUNK
        pltpu.sync_copy(idx_hbm.at[pl.ds(off, GCHUNK)], idx_v)
        row_cp = pltpu.async_copy(table_hbm.at[idx_v], rows_v, sem)
        msk_cp = pltpu.async_copy(mask_hbm.at[idx_v], mval_v, sem2)
        row_cp.wait()
        msk_cp.wait()
        pltpu.sync_copy(rows_v, emb_out.at[pl.ds(off, GCHUNK)])
        pltpu.sync_copy(mval_v, mask_out.at[pl.ds(off, GCHUNK)])


@functools.cache
def _sc_gather():
    return pl.kernel(
        _sc_gather_body,
        out_type=(
            jax.ShapeDtypeStruct((ROWS, D), jnp.float32),
            jax.ShapeDtypeStruct((ROWS,), jnp.int32),
        ),
        mesh=plsc.VectorSubcoreMesh(
            core_axis_name="c", subcore_axis_name="s",
            num_cores=NC, num_subcores=NS,
        ),
        scratch_types=[
            pltpu.VMEM((GCHUNK,), jnp.int32),
            pltpu.VMEM((GCHUNK, D), jnp.float32),
            pltpu.VMEM((GCHUNK,), jnp.int32),
            pltpu.SemaphoreType.DMA,
            pltpu.SemaphoreType.DMA,
        ],
        compiler_params=pltpu.CompilerParams(use_tc_tiling_on_sc=True),
    )


def _mlp_body(e_ref, q0_ref, q1_ref, m_ref, w1a_ref, w1b_ref, w2a_ref, w2b_ref,
              b1_ref, b2_ref, o0_ref, o1_ref, x_ref, w1s, w2s):
    bs = q0_ref.shape[0]

    @pl.when(pl.program_id(0) == 0)
    def _pack():
        zb = jnp.zeros((RANK, HID), jnp.bfloat16)
        w1s[0:RANK, 0:HID] = w1a_ref[0:RANK, :].astype(jnp.bfloat16)
        w1s[RANK:D, 0:HID] = zb
        w1s[D:2 * D, 0:HID] = w1a_ref[RANK:3 * RANK, :].astype(jnp.bfloat16)
        w1s[0:RANK, HID:2 * HID] = zb
        w1s[RANK:D, HID:2 * HID] = w1b_ref[0:RANK, :].astype(jnp.bfloat16)
        w1s[D:2 * D, HID:2 * HID] = w1b_ref[RANK:3 * RANK, :].astype(jnp.bfloat16)
        zc = jnp.zeros((HID, RANK), jnp.bfloat16)
        w2s[0:HID, 0:RANK] = w2a_ref[...].astype(jnp.bfloat16)
        w2s[0:HID, RANK:D] = zc
        w2s[HID:2 * HID, 0:RANK] = zc
        w2s[HID:2 * HID, RANK:D] = w2b_ref[...].astype(jnp.bfloat16)

    x_ref[:, 0:D] = e_ref[...].astype(jnp.bfloat16)
    for j in range(bs):
        x_ref[pl.ds(j * N, N), D:D + RANK] = q0_ref[j, :, :].astype(jnp.bfloat16)
        x_ref[pl.ds(j * N, N), D + RANK:2 * D] = (
            q1_ref[j, :, :].astype(jnp.bfloat16))
    h = jnp.dot(x_ref[...], w1s[...], preferred_element_type=jnp.float32)
    h = jnp.maximum(h + b1_ref[...], 0.0).astype(jnp.bfloat16)
    y = jnp.dot(h, w2s[...], preferred_element_type=jnp.float32)
    y = y + b2_ref[...]
    nonleaf = m_ref[...] == 0
    out = jnp.where(nonleaf, y, e_ref[...])
    for j in range(bs):
        o0_ref[j, :, :] = out[j * N:(j + 1) * N, 0:RANK]
        o1_ref[j, :, :] = out[j * N:(j + 1) * N, RANK:D]


def _run_mlp(embeds, q0, q1, mask2d, w1a, w1b, w2a, w2b, b1p, b2p, bs=8):
    grid = B // bs
    r = bs * N
    xin = 2 * D
    res = lambda i: (0, 0)
    return pl.pallas_call(
        _mlp_body,
        grid=(grid,),
        in_specs=[
            pl.BlockSpec((r, D), lambda i: (i, 0)),
            pl.BlockSpec((bs, N, RANK), lambda i: (i, 0, 0)),
            pl.BlockSpec((bs, N, RANK), lambda i: (i, 0, 0)),
            pl.BlockSpec((r, 1), lambda i: (i, 0)),
            pl.BlockSpec((3 * RANK, HID), res),
            pl.BlockSpec((3 * RANK, HID), res),
            pl.BlockSpec((HID, RANK), res),
            pl.BlockSpec((HID, RANK), res),
            pl.BlockSpec((1, 2 * HID), res),
            pl.BlockSpec((1, D), res),
        ],
        out_specs=[
            pl.BlockSpec((bs, N, RANK), lambda i: (i, 0, 0)),
            pl.BlockSpec((bs, N, RANK), lambda i: (i, 0, 0)),
        ],
        out_shape=[
            jax.ShapeDtypeStruct((B, N, RANK), jnp.float32),
            jax.ShapeDtypeStruct((B, N, RANK), jnp.float32),
        ],
        scratch_shapes=[
            pltpu.VMEM((r, xin), jnp.bfloat16),
            pltpu.VMEM((xin, 2 * HID), jnp.bfloat16),
            pltpu.VMEM((2 * HID, D), jnp.bfloat16),
        ],
    )(embeds, q0, q1, mask2d, w1a, w1b, w2a, w2b, b1p, b2p)


def kernel(nodeIdx, leaf_mask, q0, q1, table,
           qt0_w1, qt0_b1, qt0_w2, qt0_b2, qt1_w1, qt1_b1, qt1_w2, qt1_b2):
    flat_idx = nodeIdx.reshape(ROWS)
    embeds, mask = _sc_gather()(flat_idx, table, leaf_mask)
    b1p = jnp.concatenate([qt0_b1, qt1_b1]).reshape(1, 2 * HID)
    b2p = jnp.concatenate([qt0_b2, qt1_b2]).reshape(1, D)
    out0, out1 = _run_mlp(embeds, q0, q1, mask.reshape(ROWS, 1),
                          qt0_w1, qt1_w1, qt0_w2, qt1_w2, b1p, b2p)
    return out0, out1


# speedup vs baseline: 1.2853x; 1.0695x over previous
import functools

import jax
import jax.numpy as jnp
from jax import lax
from jax.experimental import pallas as pl
from jax.experimental.pallas import tpu as pltpu
from jax.experimental.pallas import tpu_sc as plsc

RANK = 64
CHUNKS = 2
NUM_NODES = 262144
B = 256
N = 128
ROWS = B * N
D = RANK * CHUNKS
HID = 10 * RANK

NC = 2
NS = 16
NW = NC * NS
ROWS_PER_W = ROWS // NW
GCHUNK = 512
N_GCHUNKS = ROWS_PER_W // GCHUNK


def _sc_gather_body(idx_hbm, table_hbm, mask_hbm, emb_out, mask_out,
                    idx_v, rows_v, mval_v, sem, sem2):
    wid = lax.axis_index("s") * NC + lax.axis_index("c")
    base = wid * ROWS_PER_W
    for c in range(N_GCHUNKS):
        off = base + c * GCHUNK
        pltpu.sync_copy(idx_hbm.at[pl.ds(off, GCHUNK)], idx_v)
        row_cp = pltpu.async_copy(table_hbm.at[idx_v], rows_v, sem)
        msk_cp = pltpu.async_copy(mask_hbm.at[idx_v], mval_v, sem2)
        row_cp.wait()
        msk_cp.wait()
        pltpu.sync_copy(rows_v, emb_out.at[pl.ds(off, GCHUNK)])
        pltpu.sync_copy(mval_v, mask_out.at[pl.ds(off, GCHUNK)])


@functools.cache
def _sc_gather():
    return pl.kernel(
        _sc_gather_body,
        out_type=(
            jax.ShapeDtypeStruct((ROWS, D), jnp.float32),
            jax.ShapeDtypeStruct((ROWS,), jnp.int32),
        ),
        mesh=plsc.VectorSubcoreMesh(
            core_axis_name="c", subcore_axis_name="s",
            num_cores=NC, num_subcores=NS,
        ),
        scratch_types=[
            pltpu.VMEM((GCHUNK,), jnp.int32),
            pltpu.VMEM((GCHUNK, D), jnp.float32),
            pltpu.VMEM((GCHUNK,), jnp.int32),
            pltpu.SemaphoreType.DMA,
            pltpu.SemaphoreType.DMA,
        ],
        compiler_params=pltpu.CompilerParams(use_tc_tiling_on_sc=True),
    )


def _mlp_body(e_ref, q0_ref, q1_ref, m_ref, w1a_ref, w1b_ref, w2a_ref, w2b_ref,
              b1_ref, b2_ref, o0_ref, o1_ref, x_ref, w1s, w2s):
    bs = q0_ref.shape[0]

    @pl.when(pl.program_id(0) == 0)
    def _pack():
        zb = jnp.zeros((RANK, HID), jnp.bfloat16)
        w1s[0:RANK, 0:HID] = w1a_ref[0:RANK, :].astype(jnp.bfloat16)
        w1s[RANK:D, 0:HID] = zb
        w1s[D:2 * D, 0:HID] = w1a_ref[RANK:3 * RANK, :].astype(jnp.bfloat16)
        w1s[0:RANK, HID:2 * HID] = zb
        w1s[RANK:D, HID:2 * HID] = w1b_ref[0:RANK, :].astype(jnp.bfloat16)
        w1s[D:2 * D, HID:2 * HID] = w1b_ref[RANK:3 * RANK, :].astype(jnp.bfloat16)
        zc = jnp.zeros((HID, RANK), jnp.bfloat16)
        w2s[0:HID, 0:RANK] = w2a_ref[...].astype(jnp.bfloat16)
        w2s[0:HID, RANK:D] = zc
        w2s[HID:2 * HID, 0:RANK] = zc
        w2s[HID:2 * HID, RANK:D] = w2b_ref[...].astype(jnp.bfloat16)

    x_ref[:, 0:D] = e_ref[...].astype(jnp.bfloat16)
    for j in range(bs):
        x_ref[pl.ds(j * N, N), D:D + RANK] = q0_ref[j, :, :].astype(jnp.bfloat16)
        x_ref[pl.ds(j * N, N), D + RANK:2 * D] = (
            q1_ref[j, :, :].astype(jnp.bfloat16))
    h = jnp.dot(x_ref[...], w1s[...], preferred_element_type=jnp.float32)
    h = jnp.maximum(h + b1_ref[...], 0.0).astype(jnp.bfloat16)
    y = jnp.dot(h, w2s[...], preferred_element_type=jnp.float32)
    y = y + b2_ref[...]
    nonleaf = m_ref[...] == 0
    out = jnp.where(nonleaf, y, e_ref[...])
    for j in range(bs):
        o0_ref[j, :, :] = out[j * N:(j + 1) * N, 0:RANK]
        o1_ref[j, :, :] = out[j * N:(j + 1) * N, RANK:D]


def _run_mlp(embeds, q0, q1, mask2d, w1a, w1b, w2a, w2b, b1p, b2p, bs=16):
    grid = B // bs
    r = bs * N
    xin = 2 * D
    res = lambda i: (0, 0)
    return pl.pallas_call(
        _mlp_body,
        grid=(grid,),
        in_specs=[
            pl.BlockSpec((r, D), lambda i: (i, 0)),
            pl.BlockSpec((bs, N, RANK), lambda i: (i, 0, 0)),
            pl.BlockSpec((bs, N, RANK), lambda i: (i, 0, 0)),
            pl.BlockSpec((r, 1), lambda i: (i, 0)),
            pl.BlockSpec((3 * RANK, HID), res),
            pl.BlockSpec((3 * RANK, HID), res),
            pl.BlockSpec((HID, RANK), res),
            pl.BlockSpec((HID, RANK), res),
            pl.BlockSpec((1, 2 * HID), res),
            pl.BlockSpec((1, D), res),
        ],
        out_specs=[
            pl.BlockSpec((bs, N, RANK), lambda i: (i, 0, 0)),
            pl.BlockSpec((bs, N, RANK), lambda i: (i, 0, 0)),
        ],
        out_shape=[
            jax.ShapeDtypeStruct((B, N, RANK), jnp.float32),
            jax.ShapeDtypeStruct((B, N, RANK), jnp.float32),
        ],
        scratch_shapes=[
            pltpu.VMEM((r, xin), jnp.bfloat16),
            pltpu.VMEM((xin, 2 * HID), jnp.bfloat16),
            pltpu.VMEM((2 * HID, D), jnp.bfloat16),
        ],
    )(embeds, q0, q1, mask2d, w1a, w1b, w2a, w2b, b1p, b2p)


def kernel(nodeIdx, leaf_mask, q0, q1, table,
           qt0_w1, qt0_b1, qt0_w2, qt0_b2, qt1_w1, qt1_b1, qt1_w2, qt1_b2):
    flat_idx = nodeIdx.reshape(ROWS)
    embeds, mask = _sc_gather()(flat_idx, table, leaf_mask)
    b1p = jnp.concatenate([qt0_b1, qt1_b1]).reshape(1, 2 * HID)
    b2p = jnp.concatenate([qt0_b2, qt1_b2]).reshape(1, D)
    out0, out1 = _run_mlp(embeds, q0, q1, mask.reshape(ROWS, 1),
                          qt0_w1, qt1_w1, qt0_w2, qt1_w2, b1p, b2p)
    return out0, out1


# speedup vs baseline: 1.2998x; 1.0113x over previous
import functools

import jax
import jax.numpy as jnp
from jax import lax
from jax.experimental import pallas as pl
from jax.experimental.pallas import tpu as pltpu
from jax.experimental.pallas import tpu_sc as plsc

RANK = 64
CHUNKS = 2
NUM_NODES = 262144
B = 256
N = 128
ROWS = B * N
D = RANK * CHUNKS
HID = 10 * RANK

NC = 2
NS = 16
NW = NC * NS
ROWS_PER_W = ROWS // NW
GCHUNK = 512
N_GCHUNKS = ROWS_PER_W // GCHUNK


def _sc_gather_body(idx_hbm, table_hbm, mask_hbm, emb_out, mask_out,
                    idx_v, rows_v, mval_v, sem, sem2):
    wid = lax.axis_index("s") * NC + lax.axis_index("c")
    base = wid * ROWS_PER_W
    for c in range(N_GCHUNKS):
        off = base + c * GCHUNK
        pltpu.sync_copy(idx_hbm.at[pl.ds(off, GCHUNK)], idx_v)
        row_cp = pltpu.async_copy(table_hbm.at[idx_v], rows_v, sem)
        msk_cp = pltpu.async_copy(mask_hbm.at[idx_v], mval_v, sem2)
        row_cp.wait()
        msk_cp.wait()
        pltpu.sync_copy(rows_v, emb_out.at[pl.ds(off, GCHUNK)])
        pltpu.sync_copy(mval_v, mask_out.at[pl.ds(off, GCHUNK)])


@functools.cache
def _sc_gather():
    return pl.kernel(
        _sc_gather_body,
        out_type=(
            jax.ShapeDtypeStruct((ROWS, D), jnp.float32),
            jax.ShapeDtypeStruct((ROWS,), jnp.int32),
        ),
        mesh=plsc.VectorSubcoreMesh(
            core_axis_name="c", subcore_axis_name="s",
            num_cores=NC, num_subcores=NS,
        ),
        scratch_types=[
            pltpu.VMEM((GCHUNK,), jnp.int32),
            pltpu.VMEM((GCHUNK, D), jnp.float32),
            pltpu.VMEM((GCHUNK,), jnp.int32),
            pltpu.SemaphoreType.DMA,
            pltpu.SemaphoreType.DMA,
        ],
        compiler_params=pltpu.CompilerParams(use_tc_tiling_on_sc=True),
    )


def _mlp_body(e_ref, q0_ref, q1_ref, m_ref, w1a_ref, w1b_ref, w2a_ref, w2b_ref,
              b1_ref, b2_ref, o0_ref, o1_ref, x_ref, w1s, w2s):
    bs = q0_ref.shape[0]

    @pl.when(pl.program_id(0) == 0)
    def _pack():
        zb = jnp.zeros((RANK, HID), jnp.bfloat16)
        w1s[0:RANK, 0:HID] = w1a_ref[0:RANK, :].astype(jnp.bfloat16)
        w1s[RANK:D, 0:HID] = zb
        w1s[D:2 * D, 0:HID] = w1a_ref[RANK:3 * RANK, :].astype(jnp.bfloat16)
        w1s[0:RANK, HID:2 * HID] = zb
        w1s[RANK:D, HID:2 * HID] = w1b_ref[0:RANK, :].astype(jnp.bfloat16)
        w1s[D:2 * D, HID:2 * HID] = w1b_ref[RANK:3 * RANK, :].astype(jnp.bfloat16)
        zc = jnp.zeros((HID, RANK), jnp.bfloat16)
        w2s[0:HID, 0:RANK] = w2a_ref[...].astype(jnp.bfloat16)
        w2s[0:HID, RANK:D] = zc
        w2s[HID:2 * HID, 0:RANK] = zc
        w2s[HID:2 * HID, RANK:D] = w2b_ref[...].astype(jnp.bfloat16)

    x_ref[:, 0:D] = e_ref[...].astype(jnp.bfloat16)
    for j in range(bs):
        x_ref[pl.ds(j * N, N), D:D + RANK] = q0_ref[j, :, :].astype(jnp.bfloat16)
        x_ref[pl.ds(j * N, N), D + RANK:2 * D] = (
            q1_ref[j, :, :].astype(jnp.bfloat16))
    h = jnp.dot(x_ref[...], w1s[...], preferred_element_type=jnp.float32)
    h = jnp.maximum(h + b1_ref[...], 0.0).astype(jnp.bfloat16)
    y = jnp.dot(h, w2s[...], preferred_element_type=jnp.float32)
    y = y + b2_ref[...]
    nonleaf = m_ref[...] == 0
    out = jnp.where(nonleaf, y, e_ref[...])
    for j in range(bs):
        o0_ref[j, :, :] = out[j * N:(j + 1) * N, 0:RANK]
        o1_ref[j, :, :] = out[j * N:(j + 1) * N, RANK:D]


def _run_mlp(embeds, q0, q1, mask2d, w1a, w1b, w2a, w2b, b1p, b2p, bs=32):
    grid = B // bs
    r = bs * N
    xin = 2 * D
    res = lambda i: (0, 0)
    return pl.pallas_call(
        _mlp_body,
        grid=(grid,),
        in_specs=[
            pl.BlockSpec((r, D), lambda i: (i, 0)),
            pl.BlockSpec((bs, N, RANK), lambda i: (i, 0, 0)),
            pl.BlockSpec((bs, N, RANK), lambda i: (i, 0, 0)),
            pl.BlockSpec((r, 1), lambda i: (i, 0)),
            pl.BlockSpec((3 * RANK, HID), res),
            pl.BlockSpec((3 * RANK, HID), res),
            pl.BlockSpec((HID, RANK), res),
            pl.BlockSpec((HID, RANK), res),
            pl.BlockSpec((1, 2 * HID), res),
            pl.BlockSpec((1, D), res),
        ],
        out_specs=[
            pl.BlockSpec((bs, N, RANK), lambda i: (i, 0, 0)),
            pl.BlockSpec((bs, N, RANK), lambda i: (i, 0, 0)),
        ],
        out_shape=[
            jax.ShapeDtypeStruct((B, N, RANK), jnp.float32),
            jax.ShapeDtypeStruct((B, N, RANK), jnp.float32),
        ],
        scratch_shapes=[
            pltpu.VMEM((r, xin), jnp.bfloat16),
            pltpu.VMEM((xin, 2 * HID), jnp.bfloat16),
            pltpu.VMEM((2 * HID, D), jnp.bfloat16),
        ],
    )(embeds, q0, q1, mask2d, w1a, w1b, w2a, w2b, b1p, b2p)


def kernel(nodeIdx, leaf_mask, q0, q1, table,
           qt0_w1, qt0_b1, qt0_w2, qt0_b2, qt1_w1, qt1_b1, qt1_w2, qt1_b2):
    flat_idx = nodeIdx.reshape(ROWS)
    embeds, mask = _sc_gather()(flat_idx, table, leaf_mask)
    b1p = jnp.concatenate([qt0_b1, qt1_b1]).reshape(1, 2 * HID)
    b2p = jnp.concatenate([qt0_b2, qt1_b2]).reshape(1, D)
    out0, out1 = _run_mlp(embeds, q0, q1, mask.reshape(ROWS, 1),
                          qt0_w1, qt1_w1, qt0_w2, qt1_w2, b1p, b2p)
    return out0, out1
